# baseline (speedup 1.0000x reference)
import math

import jax
import jax.numpy as jnp
import numpy as np
from jax import lax
from jax._src import config as _jax_config
from jax.experimental import pallas as pl
from jax.experimental.pallas import tpu as pltpu
from jax.experimental.pallas import tpu_sc as plsc

L = 16
F = 2
T = 524288
N_MIN, N_MAX = 16, 2048
_b = math.exp((math.log(N_MAX) - math.log(N_MIN)) / (L - 1))
RESOLUTIONS = [math.floor(N_MIN * _b ** i) for i in range(L)]
P1 = np.int32(np.array(2654435761, np.uint64).astype(np.uint32).view(np.int32))
P2 = np.int32(805459861)
MASK = np.int32(T - 1)

N_PTS = 262144
NC, NS = 2, 16
NW = NC * NS
PTS_PER_W = N_PTS // NW
GRP = 16
BLK = 512
NG = BLK // GRP
NBLK = PTS_PER_W // BLK
NCORNER = 8
GIDX = L * NCORNER * GRP


def _encode_kernel(xt_hbm, emb_hbm, out_hbm, x_v, out_v,
                   idx0_v, idx1_v, rows0_v, rows1_v, sem0, sem1):
    wid = (lax.axis_index("s").astype(jnp.int32) * jnp.int32(NC)
           + lax.axis_index("c").astype(jnp.int32))
    pbase = wid * jnp.int32(PTS_PER_W)
    lanes = lax.iota(jnp.int32, GRP)
    col0 = jnp.zeros((GRP,), jnp.int32)
    col1 = jnp.ones((GRP,), jnp.int32)

    def _compute_idx(g, idx_v):
        goff = g * jnp.int32(GRP)
        px = x_v[pl.ds(goff, GRP)]
        py = x_v[pl.ds(jnp.int32(BLK) + goff, GRP)]
        pz = x_v[pl.ds(jnp.int32(2 * BLK) + goff, GRP)]
        for i, res in enumerate(RESOLUTIONS):
            resf = jnp.float32(res)
            hx0 = (px * resf).astype(jnp.int32)
            hx1 = hx0 + jnp.int32(1)
            iy = (py * resf).astype(jnp.int32)
            iz = (pz * resf).astype(jnp.int32)
            hy0 = iy * P1
            hy1 = hy0 + P1
            hz0 = iz * P2
            hz1 = hz0 + P2
            lvl = jnp.int32(i * T)
            for c in range(NCORNER):
                hx = hx1 if (c & 4) else hx0
                hy = hy1 if (c & 2) else hy0
                hz = hz1 if (c & 1) else hz0
                w = ((hx ^ hy ^ hz) & MASK) + lvl
                idx_v[pl.ds(i * NCORNER * GRP + c * GRP, GRP)] = w

    def _fire(idx_v, rows_v, sem):
        pltpu.async_copy(emb_hbm.at[idx_v], rows_v, sem)

    def _drain(idx_v, rows_v, sem):
        pltpu.make_async_copy(emb_hbm.at[idx_v], rows_v, sem).wait()

    def _interp(g, rows_v):
        goff = g * jnp.int32(GRP)
        px = x_v[pl.ds(goff, GRP)]
        py = x_v[pl.ds(jnp.int32(BLK) + goff, GRP)]
        pz = x_v[pl.ds(jnp.int32(2 * BLK) + goff, GRP)]
        out_rows = goff + lanes
        for i, res in enumerate(RESOLUTIONS):
            resf = jnp.float32(res)
            xs = px * resf
            ys = py * resf
            zs = pz * resf
            fx = xs - xs.astype(jnp.int32).astype(jnp.float32)
            fy = ys - ys.astype(jnp.int32).astype(jnp.float32)
            fz = zs - zs.astype(jnp.int32).astype(jnp.float32)
            ridx = [jnp.int32(i * NCORNER * GRP + c * GRP) + lanes
                    for c in range(NCORNER)]
            for f, col in ((0, col0), (1, col1)):
                v = [plsc.load_gather(rows_v, [ridx[c], col])
                     for c in range(NCORNER)]
                c00 = v[0] + (v[4] - v[0]) * fx
                c01 = v[1] + (v[5] - v[1]) * fx
                c10 = v[2] + (v[6] - v[2]) * fx
                c11 = v[3] + (v[7] - v[3]) * fx
                c0 = c00 + (c10 - c00) * fy
                c1 = c01 + (c11 - c01) * fy
                cc = c0 + (c1 - c0) * fz
                plsc.store_scatter(
                    out_v, [out_rows, jnp.full((GRP,), i * F + f,
                                               jnp.int32)], cc)

    def _block(blk, _):
        row0 = pbase + blk * jnp.int32(BLK)
        for d in range(3):
            pltpu.sync_copy(
                xt_hbm.at[pl.ds(jnp.int32(d * N_PTS) + row0, BLK)],
                x_v.at[pl.ds(d * BLK, BLK)])

        _compute_idx(jnp.int32(0), idx0_v)
        _fire(idx0_v, rows0_v, sem0)

        def _pair(it, _):
            g0 = it * jnp.int32(2)
            g1 = g0 + jnp.int32(1)
            _compute_idx(g1, idx1_v)
            _fire(idx1_v, rows1_v, sem1)
            _drain(idx0_v, rows0_v, sem0)
            _interp(g0, rows0_v)

            @pl.when(it < jnp.int32(NG // 2 - 1))
            def _tail():
                _compute_idx(g1 + jnp.int32(1), idx0_v)
                _fire(idx0_v, rows0_v, sem0)

            _drain(idx1_v, rows1_v, sem1)
            _interp(g1, rows1_v)
            return _

        lax.fori_loop(np.int32(0), np.int32(NG // 2), _pair, None)
        pltpu.sync_copy(out_v, out_hbm.at[pl.ds(row0, BLK)])
        return _

    lax.fori_loop(np.int32(0), np.int32(NBLK), _block, None)


@jax.jit
def _encode(xt, emb):
    call = pl.kernel(
        _encode_kernel,
        out_type=jax.ShapeDtypeStruct((N_PTS, L * F), jnp.float32),
        mesh=plsc.VectorSubcoreMesh(core_axis_name="c", subcore_axis_name="s",
                                    num_cores=NC, num_subcores=NS),
        scratch_types=[
            pltpu.VMEM((3 * BLK,), jnp.float32),
            pltpu.VMEM((BLK, L * F), jnp.float32),
            pltpu.VMEM((GIDX,), jnp.int32),
            pltpu.VMEM((GIDX,), jnp.int32),
            pltpu.VMEM((GIDX, F), jnp.float32),
            pltpu.VMEM((GIDX, F), jnp.float32),
            pltpu.SemaphoreType.DMA,
            pltpu.SemaphoreType.DMA,
        ],
        compiler_params=pltpu.CompilerParams(needs_layout_passes=False,
                                             use_tc_tiling_on_sc=False),
    )
    return call(xt, emb)


def kernel(x, embeddings):
    xt = x.astype(jnp.float32).T.reshape(3 * N_PTS)
    emb = embeddings.astype(jnp.float32).reshape(L * T, F)
    with _jax_config.enable_x64(False):
        return _encode(xt, emb)


# speedup vs baseline: 1.2414x; 1.2414x over previous
import math

import jax
import jax.numpy as jnp
import numpy as np
from jax import lax
from jax._src import config as _jax_config
from jax.experimental import pallas as pl
from jax.experimental.pallas import tpu as pltpu
from jax.experimental.pallas import tpu_sc as plsc

L = 16
F = 2
T = 524288
N_MIN, N_MAX = 16, 2048
_b = math.exp((math.log(N_MAX) - math.log(N_MIN)) / (L - 1))
RESOLUTIONS = [math.floor(N_MIN * _b ** i) for i in range(L)]
P1D = np.int32(np.array((2 * 2654435761) % (1 << 32), np.uint64)
               .astype(np.uint32).view(np.int32))
P2D = np.int32(2 * 805459861)
MASKD = np.int32((T - 1) << 1)

N_PTS = 262144
NC, NS = 2, 16
NW = NC * NS
PTS_PER_W = N_PTS // NW
GRP = 16
BLK = 512
NG = BLK // GRP
NBLK = PTS_PER_W // BLK
NCORNER = 8
LVL_WORDS = T * F
STAGE_WORDS = LVL_WORDS // NS
GIDX = F * NCORNER * GRP


def _encode_kernel(xt_hbm, emb_hbm, resl_hbm, out_hbm, x_v, resl_v,
                   oc0_v, oc1_v, idx0_v, idx1_v, rows0_v, rows1_v,
                   lvl_sp, sem0, sem1):
    cid = lax.axis_index("c").astype(jnp.int32)
    sub = lax.axis_index("s").astype(jnp.int32)
    wid = sub * jnp.int32(NC) + cid
    pbase = wid * jnp.int32(PTS_PER_W)
    lanes = lax.iota(jnp.int32, GRP)

    for d in range(3):
        pltpu.sync_copy(
            xt_hbm.at[pl.ds(jnp.int32(d * N_PTS) + pbase, PTS_PER_W)],
            x_v.at[pl.ds(d * PTS_PER_W, PTS_PER_W)])
    pltpu.sync_copy(resl_hbm, resl_v)

    def _coords(poff):
        px = x_v[pl.ds(poff, GRP)]
        py = x_v[pl.ds(jnp.int32(PTS_PER_W) + poff, GRP)]
        pz = x_v[pl.ds(jnp.int32(2 * PTS_PER_W) + poff, GRP)]
        return px, py, pz

    def _level(l, _):
        resv = resl_v[pl.ds(l * jnp.int32(GRP), GRP)]
        pltpu.sync_copy(
            emb_hbm.at[pl.ds(l * jnp.int32(LVL_WORDS)
                             + sub * jnp.int32(STAGE_WORDS), STAGE_WORDS)],
            lvl_sp.at[pl.ds(sub * jnp.int32(STAGE_WORDS), STAGE_WORDS)])
        plsc.subcore_barrier()

        def _compute_idx(poff, idx_v):
            px, py, pz = _coords(poff)
            hx0 = (px * resv).astype(jnp.int32)
            hx0 = hx0 + hx0
            hx1 = hx0 + jnp.int32(2)
            iy = (py * resv).astype(jnp.int32)
            iz = (pz * resv).astype(jnp.int32)
            hy0 = iy * P1D
            hy1 = hy0 + P1D
            hz0 = iz * P2D
            hz1 = hz0 + P2D
            for c in range(NCORNER):
                hx = hx1 if (c & 4) else hx0
                hy = hy1 if (c & 2) else hy0
                hz = hz1 if (c & 1) else hz0
                w0 = (hx ^ hy ^ hz) & MASKD
                idx_v[pl.ds(c * GRP, GRP)] = w0
                idx_v[pl.ds(NCORNER * GRP + c * GRP, GRP)] = (
                    w0 + jnp.int32(1))

        def _fire(idx_v, rows_v, sem):
            pltpu.async_copy(lvl_sp.at[idx_v], rows_v, sem)

        def _drain(idx_v, rows_v, sem):
            pltpu.make_async_copy(lvl_sp.at[idx_v], rows_v, sem).wait()

        def _interp(poff, goff, rows_v):
            px, py, pz = _coords(poff)
            xs = px * resv
            ys = py * resv
            zs = pz * resv
            fx = xs - xs.astype(jnp.int32).astype(jnp.float32)
            fy = ys - ys.astype(jnp.int32).astype(jnp.float32)
            fz = zs - zs.astype(jnp.int32).astype(jnp.float32)
            for f, oc_v in ((0, oc0_v), (1, oc1_v)):
                v = [rows_v[pl.ds(f * NCORNER * GRP + c * GRP, GRP)]
                     for c in range(NCORNER)]
                c00 = v[0] + (v[4] - v[0]) * fx
                c01 = v[1] + (v[5] - v[1]) * fx
                c10 = v[2] + (v[6] - v[2]) * fx
                c11 = v[3] + (v[7] - v[3]) * fx
                c0 = c00 + (c10 - c00) * fy
                c1 = c01 + (c11 - c01) * fy
                oc_v[pl.ds(goff, GRP)] = c0 + (c1 - c0) * fz

        def _block(blk, _):
            boff = blk * jnp.int32(BLK)
            _compute_idx(boff, idx0_v)
            _fire(idx0_v, rows0_v, sem0)

            def _pair(it, _):
                goff0 = it * jnp.int32(2 * GRP)
                goff1 = goff0 + jnp.int32(GRP)
                _compute_idx(boff + goff1, idx1_v)
                _fire(idx1_v, rows1_v, sem1)
                _drain(idx0_v, rows0_v, sem0)
                _interp(boff + goff0, goff0, rows0_v)

                @pl.when(it < jnp.int32(NG // 2 - 1))
                def _tail():
                    _compute_idx(boff + goff1 + jnp.int32(GRP), idx0_v)
                    _fire(idx0_v, rows0_v, sem0)

                _drain(idx1_v, rows1_v, sem1)
                _interp(boff + goff1, goff1, rows1_v)
                return _

            lax.fori_loop(np.int32(0), np.int32(NG // 2), _pair, None)
            row0 = pbase + boff
            lf = l * jnp.int32(F)
            pltpu.sync_copy(
                oc0_v, out_hbm.at[pl.ds(lf * jnp.int32(N_PTS) + row0, BLK)])
            pltpu.sync_copy(
                oc1_v, out_hbm.at[pl.ds((lf + jnp.int32(1))
                                        * jnp.int32(N_PTS) + row0, BLK)])
            return _

        lax.fori_loop(np.int32(0), np.int32(NBLK), _block, None)
        plsc.subcore_barrier()
        return _

    lax.fori_loop(np.int32(0), np.int32(L), _level, None)


@jax.jit
def _encode(xt, emb, resl):
    call = pl.kernel(
        _encode_kernel,
        out_type=jax.ShapeDtypeStruct((L * F * N_PTS,), jnp.float32),
        mesh=plsc.VectorSubcoreMesh(core_axis_name="c", subcore_axis_name="s",
                                    num_cores=NC, num_subcores=NS),
        scratch_types=[
            pltpu.VMEM((3 * PTS_PER_W,), jnp.float32),
            pltpu.VMEM((L * GRP,), jnp.float32),
            pltpu.VMEM((BLK,), jnp.float32),
            pltpu.VMEM((BLK,), jnp.float32),
            pltpu.VMEM((GIDX,), jnp.int32),
            pltpu.VMEM((GIDX,), jnp.int32),
            pltpu.VMEM((GIDX,), jnp.float32),
            pltpu.VMEM((GIDX,), jnp.float32),
            pltpu.VMEM_SHARED((LVL_WORDS,), jnp.float32),
            pltpu.SemaphoreType.DMA,
            pltpu.SemaphoreType.DMA,
        ],
        compiler_params=pltpu.CompilerParams(needs_layout_passes=False),
    )
    return call(xt, emb, resl)


_RESL = np.repeat(np.array(RESOLUTIONS, np.float32), GRP)


def kernel(x, embeddings):
    xt = x.astype(jnp.float32).T.reshape(3 * N_PTS)
    emb = embeddings.astype(jnp.float32).reshape(L * T * F)
    resl = jnp.asarray(_RESL)
    with _jax_config.enable_x64(False):
        out = _encode(xt, emb, resl)
    return out.reshape(L * F, N_PTS).T


# speedup vs baseline: 1.2616x; 1.0163x over previous
import math

import jax
import jax.numpy as jnp
import numpy as np
from jax import lax
from jax._src import config as _jax_config
from jax.experimental import pallas as pl
from jax.experimental.pallas import tpu as pltpu
from jax.experimental.pallas import tpu_sc as plsc

L = 16
F = 2
T = 524288
N_MIN, N_MAX = 16, 2048
_b = math.exp((math.log(N_MAX) - math.log(N_MIN)) / (L - 1))
RESOLUTIONS = [math.floor(N_MIN * _b ** i) for i in range(L)]
P1D = np.int32(np.array((2 * 2654435761) % (1 << 32), np.uint64)
               .astype(np.uint32).view(np.int32))
P2D = np.int32(2 * 805459861)
MASKD = np.int32((T - 1) << 1)

N_PTS = 262144
NC, NS = 2, 16
NW = NC * NS
PTS_PER_W = N_PTS // NW
GRP = 16
BLK = 512
NG = BLK // GRP
NBLK = PTS_PER_W // BLK
NCORNER = 8
LVL_WORDS = T * F
STAGE_WORDS = LVL_WORDS // NS
GIDX = F * NCORNER * GRP
GPG = 2
SGRP = GPG * GRP
NSG = BLK // SGRP
BIDX = GPG * GIDX


def _encode_kernel(xt_hbm, emb_hbm, resl_hbm, out_hbm, x_v, resl_v,
                   oc0_v, oc1_v, idx0_v, idx1_v, rows0_v, rows1_v,
                   lvl_sp, sem0, sem1):
    cid = lax.axis_index("c").astype(jnp.int32)
    sub = lax.axis_index("s").astype(jnp.int32)
    wid = sub * jnp.int32(NC) + cid
    pbase = wid * jnp.int32(PTS_PER_W)
    lanes = lax.iota(jnp.int32, GRP)

    for d in range(3):
        pltpu.sync_copy(
            xt_hbm.at[pl.ds(jnp.int32(d * N_PTS) + pbase, PTS_PER_W)],
            x_v.at[pl.ds(d * PTS_PER_W, PTS_PER_W)])
    pltpu.sync_copy(resl_hbm, resl_v)

    def _coords(poff):
        px = x_v[pl.ds(poff, GRP)]
        py = x_v[pl.ds(jnp.int32(PTS_PER_W) + poff, GRP)]
        pz = x_v[pl.ds(jnp.int32(2 * PTS_PER_W) + poff, GRP)]
        return px, py, pz

    def _level(l, _):
        resv = resl_v[pl.ds(l * jnp.int32(GRP), GRP)]
        pltpu.sync_copy(
            emb_hbm.at[pl.ds(l * jnp.int32(LVL_WORDS)
                             + sub * jnp.int32(STAGE_WORDS), STAGE_WORDS)],
            lvl_sp.at[pl.ds(sub * jnp.int32(STAGE_WORDS), STAGE_WORDS)])
        plsc.subcore_barrier()

        def _compute_idx(poff, idx_v):
            for g in range(GPG):
                px, py, pz = _coords(poff + jnp.int32(g * GRP))
                hx0 = (px * resv).astype(jnp.int32)
                hx0 = hx0 + hx0
                hx1 = hx0 + jnp.int32(2)
                iy = (py * resv).astype(jnp.int32)
                iz = (pz * resv).astype(jnp.int32)
                hy0 = iy * P1D
                hy1 = hy0 + P1D
                hz0 = iz * P2D
                hz1 = hz0 + P2D
                gb = g * GIDX
                for c in range(NCORNER):
                    hx = hx1 if (c & 4) else hx0
                    hy = hy1 if (c & 2) else hy0
                    hz = hz1 if (c & 1) else hz0
                    w0 = (hx ^ hy ^ hz) & MASKD
                    idx_v[pl.ds(gb + c * GRP, GRP)] = w0
                    idx_v[pl.ds(gb + NCORNER * GRP + c * GRP, GRP)] = (
                        w0 + jnp.int32(1))

        def _fire(idx_v, rows_v, sem):
            pltpu.async_copy(lvl_sp.at[idx_v], rows_v, sem)

        def _drain(idx_v, rows_v, sem):
            pltpu.make_async_copy(lvl_sp.at[idx_v], rows_v, sem).wait()

        def _interp(poff, goff, rows_v):
            for g in range(GPG):
                px, py, pz = _coords(poff + jnp.int32(g * GRP))
                xs = px * resv
                ys = py * resv
                zs = pz * resv
                fx = xs - xs.astype(jnp.int32).astype(jnp.float32)
                fy = ys - ys.astype(jnp.int32).astype(jnp.float32)
                fz = zs - zs.astype(jnp.int32).astype(jnp.float32)
                gb = g * GIDX
                for f, oc_v in ((0, oc0_v), (1, oc1_v)):
                    v = [rows_v[pl.ds(gb + f * NCORNER * GRP + c * GRP, GRP)]
                         for c in range(NCORNER)]
                    c00 = v[0] + (v[4] - v[0]) * fx
                    c01 = v[1] + (v[5] - v[1]) * fx
                    c10 = v[2] + (v[6] - v[2]) * fx
                    c11 = v[3] + (v[7] - v[3]) * fx
                    c0 = c00 + (c10 - c00) * fy
                    c1 = c01 + (c11 - c01) * fy
                    oc_v[pl.ds(goff + jnp.int32(g * GRP), GRP)] = (
                        c0 + (c1 - c0) * fz)

        def _block(blk, _):
            boff = blk * jnp.int32(BLK)
            _compute_idx(boff, idx0_v)
            _fire(idx0_v, rows0_v, sem0)

            def _pair(it, _):
                goff0 = it * jnp.int32(2 * SGRP)
                goff1 = goff0 + jnp.int32(SGRP)
                _compute_idx(boff + goff1, idx1_v)
                _fire(idx1_v, rows1_v, sem1)
                _drain(idx0_v, rows0_v, sem0)
                _interp(boff + goff0, goff0, rows0_v)

                @pl.when(it < jnp.int32(NSG // 2 - 1))
                def _tail():
                    _compute_idx(boff + goff1 + jnp.int32(SGRP), idx0_v)
                    _fire(idx0_v, rows0_v, sem0)

                _drain(idx1_v, rows1_v, sem1)
                _interp(boff + goff1, goff1, rows1_v)
                return _

            lax.fori_loop(np.int32(0), np.int32(NSG // 2), _pair, None)
            row0 = pbase + boff
            lf = l * jnp.int32(F)
            pltpu.sync_copy(
                oc0_v, out_hbm.at[pl.ds(lf * jnp.int32(N_PTS) + row0, BLK)])
            pltpu.sync_copy(
                oc1_v, out_hbm.at[pl.ds((lf + jnp.int32(1))
                                        * jnp.int32(N_PTS) + row0, BLK)])
            return _

        lax.fori_loop(np.int32(0), np.int32(NBLK), _block, None)
        plsc.subcore_barrier()
        return _

    lax.fori_loop(np.int32(0), np.int32(L), _level, None)


@jax.jit
def _encode(xt, emb, resl):
    call = pl.kernel(
        _encode_kernel,
        out_type=jax.ShapeDtypeStruct((L * F * N_PTS,), jnp.float32),
        mesh=plsc.VectorSubcoreMesh(core_axis_name="c", subcore_axis_name="s",
                                    num_cores=NC, num_subcores=NS),
        scratch_types=[
            pltpu.VMEM((3 * PTS_PER_W,), jnp.float32),
            pltpu.VMEM((L * GRP,), jnp.float32),
            pltpu.VMEM((BLK,), jnp.float32),
            pltpu.VMEM((BLK,), jnp.float32),
            pltpu.VMEM((BIDX,), jnp.int32),
            pltpu.VMEM((BIDX,), jnp.int32),
            pltpu.VMEM((BIDX,), jnp.float32),
            pltpu.VMEM((BIDX,), jnp.float32),
            pltpu.VMEM_SHARED((LVL_WORDS,), jnp.float32),
            pltpu.SemaphoreType.DMA,
            pltpu.SemaphoreType.DMA,
        ],
        compiler_params=pltpu.CompilerParams(needs_layout_passes=False),
    )
    return call(xt, emb, resl)


_RESL = np.repeat(np.array(RESOLUTIONS, np.float32), GRP)


def kernel(x, embeddings):
    xt = x.astype(jnp.float32).T.reshape(3 * N_PTS)
    emb = embeddings.astype(jnp.float32).reshape(L * T * F)
    resl = jnp.asarray(_RESL)
    with _jax_config.enable_x64(False):
        out = _encode(xt, emb, resl)
    return out.reshape(L * F, N_PTS).T


# speedup vs baseline: 1.2664x; 1.0037x over previous
import math

import jax
import jax.numpy as jnp
import numpy as np
from jax import lax
from jax._src import config as _jax_config
from jax.experimental import pallas as pl
from jax.experimental.pallas import tpu as pltpu
from jax.experimental.pallas import tpu_sc as plsc

L = 16
F = 2
T = 524288
N_MIN, N_MAX = 16, 2048
_b = math.exp((math.log(N_MAX) - math.log(N_MIN)) / (L - 1))
RESOLUTIONS = [math.floor(N_MIN * _b ** i) for i in range(L)]
P1D = np.int32(np.array((2 * 2654435761) % (1 << 32), np.uint64)
               .astype(np.uint32).view(np.int32))
P2D = np.int32(2 * 805459861)
MASKD = np.int32((T - 1) << 1)

N_PTS = 262144
NC, NS = 2, 16
NW = NC * NS
PTS_PER_W = N_PTS // NW
GRP = 16
BLK = 512
NG = BLK // GRP
NBLK = PTS_PER_W // BLK
NCORNER = 8
LVL_WORDS = T * F
STAGE_WORDS = LVL_WORDS // NS
GIDX = F * NCORNER * GRP
GPG = 4
SGRP = GPG * GRP
NSG = BLK // SGRP
BIDX = GPG * GIDX


def _encode_kernel(xt_hbm, emb_hbm, resl_hbm, out_hbm, x_v, resl_v,
                   oc0_v, oc1_v, idx0_v, idx1_v, rows0_v, rows1_v,
                   lvl_sp, sem0, sem1):
    cid = lax.axis_index("c").astype(jnp.int32)
    sub = lax.axis_index("s").astype(jnp.int32)
    wid = sub * jnp.int32(NC) + cid
    pbase = wid * jnp.int32(PTS_PER_W)
    lanes = lax.iota(jnp.int32, GRP)

    for d in range(3):
        pltpu.sync_copy(
            xt_hbm.at[pl.ds(jnp.int32(d * N_PTS) + pbase, PTS_PER_W)],
            x_v.at[pl.ds(d * PTS_PER_W, PTS_PER_W)])
    pltpu.sync_copy(resl_hbm, resl_v)

    def _coords(poff):
        px = x_v[pl.ds(poff, GRP)]
        py = x_v[pl.ds(jnp.int32(PTS_PER_W) + poff, GRP)]
        pz = x_v[pl.ds(jnp.int32(2 * PTS_PER_W) + poff, GRP)]
        return px, py, pz

    def _level(l, _):
        resv = resl_v[pl.ds(l * jnp.int32(GRP), GRP)]
        pltpu.sync_copy(
            emb_hbm.at[pl.ds(l * jnp.int32(LVL_WORDS)
                             + sub * jnp.int32(STAGE_WORDS), STAGE_WORDS)],
            lvl_sp.at[pl.ds(sub * jnp.int32(STAGE_WORDS), STAGE_WORDS)])
        plsc.subcore_barrier()

        def _compute_idx(poff, idx_v):
            for g in range(GPG):
                px, py, pz = _coords(poff + jnp.int32(g * GRP))
                hx0 = (px * resv).astype(jnp.int32)
                hx0 = hx0 + hx0
                hx1 = hx0 + jnp.int32(2)
                iy = (py * resv).astype(jnp.int32)
                iz = (pz * resv).astype(jnp.int32)
                hy0 = iy * P1D
                hy1 = hy0 + P1D
                hz0 = iz * P2D
                hz1 = hz0 + P2D
                gb = g * GIDX
                for c in range(NCORNER):
                    hx = hx1 if (c & 4) else hx0
                    hy = hy1 if (c & 2) else hy0
                    hz = hz1 if (c & 1) else hz0
                    w0 = (hx ^ hy ^ hz) & MASKD
                    idx_v[pl.ds(gb + c * GRP, GRP)] = w0
                    idx_v[pl.ds(gb + NCORNER * GRP + c * GRP, GRP)] = (
                        w0 + jnp.int32(1))

        def _fire(idx_v, rows_v, sem):
            pltpu.async_copy(lvl_sp.at[idx_v], rows_v, sem)

        def _drain(idx_v, rows_v, sem):
            pltpu.make_async_copy(lvl_sp.at[idx_v], rows_v, sem).wait()

        def _interp(poff, goff, rows_v):
            for g in range(GPG):
                px, py, pz = _coords(poff + jnp.int32(g * GRP))
                xs = px * resv
                ys = py * resv
                zs = pz * resv
                fx = xs - xs.astype(jnp.int32).astype(jnp.float32)
                fy = ys - ys.astype(jnp.int32).astype(jnp.float32)
                fz = zs - zs.astype(jnp.int32).astype(jnp.float32)
                gb = g * GIDX
                for f, oc_v in ((0, oc0_v), (1, oc1_v)):
                    v = [rows_v[pl.ds(gb + f * NCORNER * GRP + c * GRP, GRP)]
                         for c in range(NCORNER)]
                    c00 = v[0] + (v[4] - v[0]) * fx
                    c01 = v[1] + (v[5] - v[1]) * fx
                    c10 = v[2] + (v[6] - v[2]) * fx
                    c11 = v[3] + (v[7] - v[3]) * fx
                    c0 = c00 + (c10 - c00) * fy
                    c1 = c01 + (c11 - c01) * fy
                    oc_v[pl.ds(goff + jnp.int32(g * GRP), GRP)] = (
                        c0 + (c1 - c0) * fz)

        def _block(blk, _):
            boff = blk * jnp.int32(BLK)
            _compute_idx(boff, idx0_v)
            _fire(idx0_v, rows0_v, sem0)

            def _pair(it, _):
                goff0 = it * jnp.int32(2 * SGRP)
                goff1 = goff0 + jnp.int32(SGRP)
                _compute_idx(boff + goff1, idx1_v)
                _fire(idx1_v, rows1_v, sem1)
                _drain(idx0_v, rows0_v, sem0)
                _interp(boff + goff0, goff0, rows0_v)

                @pl.when(it < jnp.int32(NSG // 2 - 1))
                def _tail():
                    _compute_idx(boff + goff1 + jnp.int32(SGRP), idx0_v)
                    _fire(idx0_v, rows0_v, sem0)

                _drain(idx1_v, rows1_v, sem1)
                _interp(boff + goff1, goff1, rows1_v)
                return _

            lax.fori_loop(np.int32(0), np.int32(NSG // 2), _pair, None)
            row0 = pbase + boff
            lf = l * jnp.int32(F)
            pltpu.sync_copy(
                oc0_v, out_hbm.at[pl.ds(lf * jnp.int32(N_PTS) + row0, BLK)])
            pltpu.sync_copy(
                oc1_v, out_hbm.at[pl.ds((lf + jnp.int32(1))
                                        * jnp.int32(N_PTS) + row0, BLK)])
            return _

        lax.fori_loop(np.int32(0), np.int32(NBLK), _block, None)
        plsc.subcore_barrier()
        return _

    lax.fori_loop(np.int32(0), np.int32(L), _level, None)


@jax.jit
def _encode(xt, emb, resl):
    call = pl.kernel(
        _encode_kernel,
        out_type=jax.ShapeDtypeStruct((L * F * N_PTS,), jnp.float32),
        mesh=plsc.VectorSubcoreMesh(core_axis_name="c", subcore_axis_name="s",
                                    num_cores=NC, num_subcores=NS),
        scratch_types=[
            pltpu.VMEM((3 * PTS_PER_W,), jnp.float32),
            pltpu.VMEM((L * GRP,), jnp.float32),
            pltpu.VMEM((BLK,), jnp.float32),
            pltpu.VMEM((BLK,), jnp.float32),
            pltpu.VMEM((BIDX,), jnp.int32),
            pltpu.VMEM((BIDX,), jnp.int32),
            pltpu.VMEM((BIDX,), jnp.float32),
            pltpu.VMEM((BIDX,), jnp.float32),
            pltpu.VMEM_SHARED((LVL_WORDS,), jnp.float32),
            pltpu.SemaphoreType.DMA,
            pltpu.SemaphoreType.DMA,
        ],
        compiler_params=pltpu.CompilerParams(needs_layout_passes=False),
    )
    return call(xt, emb, resl)


_RESL = np.repeat(np.array(RESOLUTIONS, np.float32), GRP)


def kernel(x, embeddings):
    xt = x.astype(jnp.float32).T.reshape(3 * N_PTS)
    emb = embeddings.astype(jnp.float32).reshape(L * T * F)
    resl = jnp.asarray(_RESL)
    with _jax_config.enable_x64(False):
        out = _encode(xt, emb, resl)
    return out.reshape(L * F, N_PTS).T
